# baseline (speedup 1.0000x reference)
import functools
import math

import jax
import jax.numpy as jnp
import numpy as np
from jax import lax
from jax.experimental import pallas as pl
from jax.experimental.pallas import tpu as pltpu
from jax.experimental.pallas import tpu_sc as plsc

N_ROWS = 8192
N_COLS = 32000
PAD = 0
EPS = float(np.float32(0.1 / (N_COLS - 1)))
CONF = 0.9
C_ROW = (N_COLS - 1) * EPS * math.log(EPS) + CONF * math.log(CONF)
CME = CONF - EPS

NC = 2
NS = 16
L = 16
NW = NC * NS

SC_ROWS = 512
N_TC = N_ROWS - SC_ROWS
RPW = SC_ROWS // NW
NBUF = 2
GPW = N_TC // NW
NGRP = GPW // L

UNR = 16
INNER = N_COLS // (L * UNR)


@functools.cache
def _make_sc_kernel():
    mesh = plsc.VectorSubcoreMesh(core_axis_name="c", subcore_axis_name="s")

    @functools.partial(
        pl.kernel,
        mesh=mesh,
        out_type=jax.ShapeDtypeStruct((NW, L), jnp.float32),
        scratch_types=[
            pltpu.VMEM((GPW,), jnp.int32),
            pltpu.VMEM((GPW * L,), jnp.float32),
            pltpu.VMEM((L,), jnp.int32),
            pltpu.VMEM((NBUF, N_COLS), jnp.float32),
            pltpu.VMEM((L,), jnp.float32),
            pltpu.SemaphoreType.DMA,
            pltpu.SemaphoreType.DMA,
            pltpu.SemaphoreType.DMA,
        ],
    )
    def _sc_kernel(x2_hbm, tgt_hbm, lout_hbm,
                   tgt_g, val_g, tgt_s, row_v, lsum_v,
                   sem_g, semb0, semb1):
        wid = lax.axis_index("s") * NC + lax.axis_index("c")

        base_g = wid * GPW
        pltpu.sync_copy(tgt_hbm.at[pl.ds(base_g, GPW)], tgt_g)

        def fire(g, carry):
            tchunk = tgt_g[pl.ds(g * L, L)]
            for j in range(L):
                slot = g * L + j
                t_s = tchunk[j]
                goff = (t_s // L) * L
                pltpu.async_copy(
                    x2_hbm.at[base_g + slot, pl.ds(goff, L)],
                    val_g.at[pl.ds(slot * L, L)],
                    sem_g,
                )
            return carry

        lax.fori_loop(0, NGRP, fire, 0)

        base_s = N_TC + wid * RPW
        pltpu.sync_copy(tgt_hbm.at[pl.ds(base_s, RPW)], tgt_s)
        sems = [semb0, semb1]
        for b in range(NBUF):
            pltpu.async_copy(x2_hbm.at[base_s + b], row_v.at[b], sems[b])

        tchunk_s = tgt_s[...]
        loss_acc = jnp.zeros((L,), jnp.float32)
        for j in range(RPW):
            b = j % NBUF
            pltpu.make_async_copy(
                x2_hbm.at[0], row_v.at[b], sems[b]).wait()
            row_ref = row_v.at[b]

            def inner(i, accs):
                a0, a1, a2, a3 = accs
                o = i * (L * UNR)
                vs = [row_ref[pl.ds(o + u * L, L)] for u in range(UNR)]
                for u in range(0, UNR, 4):
                    a0 = a0 + vs[u]
                    a1 = a1 + vs[u + 1]
                    a2 = a2 + vs[u + 2]
                    a3 = a3 + vs[u + 3]
                return (a0, a1, a2, a3)

            z = jnp.zeros((L,), jnp.float32)
            a0, a1, a2, a3 = lax.fori_loop(0, INNER, inner, (z, z, z, z))
            rowtot = (a0 + a1) + (a2 + a3)
            t_s = tchunk_s[j]
            tval = jnp.full((L,), t_s, jnp.int32)
            goff = (t_s // L) * L
            gchunk = row_ref[pl.ds(goff, L)]
            gvec = gchunk.at[tval - goff].get(mode="promise_in_bounds")
            m = jnp.where(t_s != PAD, 1.0, 0.0)
            loss_acc = loss_acc + m * (
                C_ROW / L - EPS * rowtot - (CME / L) * gvec)
            nxt = jnp.minimum(base_s + j + NBUF, N_ROWS - 1)
            pltpu.async_copy(x2_hbm.at[nxt], row_v.at[b], sems[b])
        for b in range(NBUF):
            pltpu.make_async_copy(x2_hbm.at[0], row_v.at[b], sems[b]).wait()

        pltpu.make_async_copy(
            x2_hbm.at[0, pl.ds(0, GPW * L)], val_g, sem_g).wait()

        def acc_loop(g, acc):
            tchunk = tgt_g[pl.ds(g * L, L)]
            for j in range(L):
                slot = g * L + j
                t_s = tchunk[j]
                goff = (t_s // L) * L
                chunk = val_g[pl.ds(slot * L, L)]
                gvec = chunk.at[
                    jnp.full((L,), t_s - goff, jnp.int32)
                ].get(mode="promise_in_bounds")
                m = jnp.where(t_s != PAD, -CME / L, 0.0)
                acc = acc + m * gvec
            return acc

        loss_acc = lax.fori_loop(0, NGRP, acc_loop, loss_acc)
        lsum_v[...] = loss_acc
        pltpu.sync_copy(lsum_v, lout_hbm.at[wid])

    return _sc_kernel


ROW_BLK = 128
GRID = N_TC // ROW_BLK


def _tc_body(x_ref, t_ref, out_ref):
    i = pl.program_id(0)

    @pl.when(i == 0)
    def _init():
        out_ref[0, 0] = 0.0

    xb = x_ref[...]
    tb = t_ref[...]
    wb = jnp.where(tb != PAD, 1.0, 0.0).astype(jnp.float32)
    rs = jnp.sum(xb, axis=1)
    out_ref[0, 0] += jnp.sum(wb * (C_ROW - EPS * rs))


_tc_call = pl.pallas_call(
    _tc_body,
    grid=(GRID,),
    in_specs=[
        pl.BlockSpec((ROW_BLK, N_COLS), lambda i: (i, 0)),
        pl.BlockSpec((ROW_BLK,), lambda i: (i,)),
    ],
    out_specs=pl.BlockSpec(
        (1, 1), lambda i: (0, 0), memory_space=pltpu.SMEM
    ),
    out_shape=jax.ShapeDtypeStruct((1, 1), jnp.float32),
)


def _combine_body(dense_ref, l_ref, out_ref):
    out_ref[0, 0] = dense_ref[0, 0] + jnp.sum(l_ref[...])


_combine_call = pl.pallas_call(
    _combine_body,
    in_specs=[
        pl.BlockSpec(memory_space=pltpu.SMEM),
        pl.BlockSpec((NW, L), lambda: (0, 0)),
    ],
    out_specs=pl.BlockSpec(memory_space=pltpu.SMEM),
    out_shape=jax.ShapeDtypeStruct((1, 1), jnp.float32),
)


def kernel(x, target):
    tgt = target.astype(jnp.int32)
    lparts = _make_sc_kernel()(x, tgt)
    dense = _tc_call(x, tgt)
    out = _combine_call(dense, lparts)
    return out[0, 0]


# speedup vs baseline: 1.0128x; 1.0128x over previous
import functools
import math

import jax
import jax.numpy as jnp
import numpy as np
from jax import lax
from jax.experimental import pallas as pl
from jax.experimental.pallas import tpu as pltpu
from jax.experimental.pallas import tpu_sc as plsc

N_ROWS = 8192
N_COLS = 32000
PAD = 0
EPS = float(np.float32(0.1 / (N_COLS - 1)))
CONF = 0.9
C_ROW = (N_COLS - 1) * EPS * math.log(EPS) + CONF * math.log(CONF)
CME = CONF - EPS

NC = 2
NS = 16
L = 16
NW = NC * NS
PER_W = N_ROWS // NW
NGRP = PER_W // L


@functools.cache
def _make_sc_kernel():
    mesh = plsc.VectorSubcoreMesh(core_axis_name="c", subcore_axis_name="s")

    @functools.partial(
        pl.kernel,
        mesh=mesh,
        out_type=jax.ShapeDtypeStruct((NW, L), jnp.float32),
        scratch_types=[
            pltpu.VMEM((PER_W,), jnp.int32),
            pltpu.VMEM((PER_W * L,), jnp.float32),
            pltpu.VMEM((L,), jnp.float32),
            pltpu.SemaphoreType.DMA,
        ],
    )
    def _sc_kernel(x2_hbm, tgt_hbm, gout_hbm, tgt_v, val_v, acc_v, sem):
        wid = lax.axis_index("s") * NC + lax.axis_index("c")
        base = wid * PER_W
        pltpu.sync_copy(tgt_hbm.at[pl.ds(base, PER_W)], tgt_v)

        def fire(g, carry):
            tchunk = tgt_v[pl.ds(g * L, L)]
            for j in range(L):
                slot = g * L + j
                t_s = tchunk[j]
                goff = (t_s // L) * L
                pltpu.async_copy(
                    x2_hbm.at[base + slot, pl.ds(goff, L)],
                    val_v.at[pl.ds(slot * L, L)],
                    sem,
                )
            return carry

        lax.fori_loop(0, NGRP, fire, 0)
        pltpu.make_async_copy(
            x2_hbm.at[0, pl.ds(0, PER_W * L)], val_v, sem).wait()

        def acc_loop(g, acc):
            tchunk = tgt_v[pl.ds(g * L, L)]
            for j in range(L):
                slot = g * L + j
                t_s = tchunk[j]
                goff = (t_s // L) * L
                chunk = val_v[pl.ds(slot * L, L)]
                gvec = chunk.at[
                    jnp.full((L,), t_s - goff, jnp.int32)
                ].get(mode="promise_in_bounds")
                m = jnp.where(t_s != PAD, 1.0 / L, 0.0)
                acc = acc + m * gvec
            return acc

        acc = lax.fori_loop(0, NGRP, acc_loop, jnp.zeros((L,), jnp.float32))
        acc_v[...] = acc
        pltpu.sync_copy(acc_v, gout_hbm.at[wid])

    return _sc_kernel


ROW_BLK = 128
GRID = N_ROWS // ROW_BLK


def _tc_body(x_ref, t_ref, out_ref):
    i = pl.program_id(0)

    @pl.when(i == 0)
    def _init():
        out_ref[0, 0] = 0.0

    xb = x_ref[...]
    tb = t_ref[...]
    wb = jnp.where(tb != PAD, 1.0, 0.0).astype(jnp.float32)
    rs = jnp.sum(xb, axis=1)
    out_ref[0, 0] += jnp.sum(wb * (C_ROW - EPS * rs))


_tc_call = pl.pallas_call(
    _tc_body,
    grid=(GRID,),
    in_specs=[
        pl.BlockSpec((ROW_BLK, N_COLS), lambda i: (i, 0)),
        pl.BlockSpec((ROW_BLK,), lambda i: (i,)),
    ],
    out_specs=pl.BlockSpec(
        (1, 1), lambda i: (0, 0), memory_space=pltpu.SMEM
    ),
    out_shape=jax.ShapeDtypeStruct((1, 1), jnp.float32),
)


def _combine_body(dense_ref, g_ref, out_ref):
    out_ref[0, 0] = dense_ref[0, 0] - CME * jnp.sum(g_ref[...])


_combine_call = pl.pallas_call(
    _combine_body,
    in_specs=[
        pl.BlockSpec(memory_space=pltpu.SMEM),
        pl.BlockSpec((NW, L), lambda: (0, 0)),
    ],
    out_specs=pl.BlockSpec(memory_space=pltpu.SMEM),
    out_shape=jax.ShapeDtypeStruct((1, 1), jnp.float32),
)


def kernel(x, target):
    tgt = target.astype(jnp.int32)
    gparts = _make_sc_kernel()(x, tgt)
    dense = _tc_call(x, tgt)
    out = _combine_call(dense, gparts)
    return out[0, 0]
